# baseline (speedup 1.0000x reference)
import functools

import jax
import jax.numpy as jnp
from jax import lax
from jax.experimental import pallas as pl
from jax.experimental.pallas import tpu as pltpu
from jax.experimental.pallas import tpu_sc as plsc

HP = 384
PH = 128
ONES_COL = 310
ONES_LOCAL = ONES_COL - 2 * PH
NS = 16
BLK = 2000
OUT_PAD = 128


def _mm(a, b):
  return lax.dot_general(a, b, (((1,), (0,)), ((), ())),
                         preferred_element_type=jnp.float32,
                         precision=lax.Precision.HIGHEST)


def _set_ones_col(m):
  col = lax.broadcasted_iota(jnp.int32, m.shape, 1)
  return jnp.where(col == ONES_COL, 1.0, m)


def _lift_body(x_ref, wl_ref, bl_ref, wm_ref, bm_ref,
               h_ref, m0_ref, m1_ref, m2_ref):
  h = _mm(x_ref[...], wl_ref[...]) + bl_ref[...]
  h_ref[...] = h
  m = jnp.maximum(_mm(h, wm_ref[...]) + bm_ref[...], 0.0)
  m = _set_ones_col(m)
  m0_ref[...] = m[:, :PH]
  m1_ref[...] = m[:, PH:2 * PH]
  m2_ref[...] = m[:, 2 * PH:]


def _mp_body(a0_ref, a1a_ref, a1b_ref, a2_ref, hp_ref, wo_ref, bo_ref,
             wm_ref, bm_ref, h_ref, m0_ref, m1_ref, m2_ref):
  deg = a2_ref[:, ONES_LOCAL:ONES_LOCAL + 1]
  a1 = a1a_ref[...] + a1b_ref[...]
  feats = jnp.concatenate([a0_ref[...], a1, a2_ref[...]], axis=1)
  feats = jnp.where(deg > 0.0, feats, hp_ref[...])
  h = jnp.maximum(_mm(feats, wo_ref[...]) + bo_ref[...], 0.0)
  h_ref[...] = h
  m = jnp.maximum(_mm(h, wm_ref[...]) + bm_ref[...], 0.0)
  m = _set_ones_col(m)
  m0_ref[...] = m[:, :PH]
  m1_ref[...] = m[:, PH:2 * PH]
  m2_ref[...] = m[:, 2 * PH:]


def _final_body(a0_ref, a1a_ref, a1b_ref, a2_ref, hp_ref, wo_ref, bo_ref,
                wr_ref, br_ref, gid_ref, out_ref):
  i = pl.program_id(0)
  deg = a2_ref[:, ONES_LOCAL:ONES_LOCAL + 1]
  a1 = a1a_ref[...] + a1b_ref[...]
  feats = jnp.concatenate([a0_ref[...], a1, a2_ref[...]], axis=1)
  feats = jnp.where(deg > 0.0, feats, hp_ref[...])
  h = jnp.maximum(_mm(feats, wo_ref[...]) + bo_ref[...], 0.0)
  lg = _mm(h, wr_ref[...]) + br_ref[...]
  col = lax.broadcasted_iota(jnp.int32, lg.shape, 1)
  is_cls = col < 2
  mx = jnp.max(jnp.where(is_cls, lg, -1e30), axis=1, keepdims=True)
  ex = jnp.where(is_cls, jnp.exp(lg - mx), 0.0)
  p = ex / jnp.sum(ex, axis=1, keepdims=True)
  gid = gid_ref[0, 0, :]
  onehot = (gid[:, None] == col).astype(jnp.float32)
  contrib = lax.dot_general(onehot, p, (((0,), (0,)), ((), ())),
                            preferred_element_type=jnp.float32,
                            precision=lax.Precision.HIGHEST)

  @pl.when(i == 0)
  def _():
    out_ref[...] = jnp.zeros_like(out_ref)

  out_ref[...] += contrib


def _row_spec(blk, w):
  return pl.BlockSpec((blk, w), lambda i: (i, 0))


def _full_spec(r, c):
  return pl.BlockSpec((r, c), lambda i: (0, 0))


def _node_outs(n):
  return (
      [_row_spec(BLK, HP)] + [_row_spec(BLK, PH)] * 3,
      [jax.ShapeDtypeStruct((n, HP), jnp.float32)] +
      [jax.ShapeDtypeStruct((n, PH), jnp.float32)] * 3,
  )


def _make_lift(n, din):
  out_specs, out_shape = _node_outs(n)
  return pl.pallas_call(
      _lift_body,
      grid=(n // BLK,),
      in_specs=[_row_spec(BLK, din), _full_spec(din, HP), _full_spec(1, HP),
                _full_spec(HP, HP), _full_spec(1, HP)],
      out_specs=out_specs,
      out_shape=out_shape,
  )


def _make_mp(n):
  out_specs, out_shape = _node_outs(n)
  return pl.pallas_call(
      _mp_body,
      grid=(n // BLK,),
      in_specs=[_row_spec(BLK, PH)] * 4 + [_row_spec(BLK, HP),
                _full_spec(HP, HP), _full_spec(1, HP),
                _full_spec(HP, HP), _full_spec(1, HP)],
      out_specs=out_specs,
      out_shape=out_shape,
  )


def _make_final(n):
  return pl.pallas_call(
      _final_body,
      grid=(n // BLK,),
      in_specs=[_row_spec(BLK, PH)] * 4 + [_row_spec(BLK, HP),
                _full_spec(HP, HP), _full_spec(1, HP),
                _full_spec(HP, OUT_PAD), _full_spec(1, OUT_PAD),
                pl.BlockSpec((1, 1, BLK), lambda i: (i, 0, 0))],
      out_specs=pl.BlockSpec((OUT_PAD, OUT_PAD), lambda i: (0, 0)),
      out_shape=jax.ShapeDtypeStruct((OUT_PAD, OUT_PAD), jnp.float32),
  )


def _pad_rows(n):
  return ((n + 8 * NS - 1) // (8 * NS)) * (8 * NS)


K = 88
NBUF = 4
IBUF = 6
U = 12
SLAG = NBUF - 1
ILOOK = IBUF - SLAG


def _sc_chunks(e):
  ew = e // NS
  ch = -(-ew // K)
  ch = -(-ch // (2 * U)) * (2 * U)
  return ew, ch


def _make_sc_scatter(n, e):
  ew, ch = _sc_chunks(e)
  ngrp = ch // U
  np_ = _pad_rows(n)
  rp = np_ // NS

  mesh = plsc.VectorSubcoreMesh(core_axis_name="c", subcore_axis_name="s")

  @functools.partial(
      pl.kernel,
      out_type=(jax.ShapeDtypeStruct((np_, PH), jnp.float32),) * 4,
      mesh=mesh,
      scratch_types=[
          pltpu.VMEM((IBUF, 2, K), jnp.int32),
          [pltpu.VMEM((K, PH), jnp.float32) for _ in range(NBUF)],
          pltpu.VMEM_SHARED((np_, PH), jnp.float32),
          pltpu.SemaphoreType.DMA((IBUF,)),
          pltpu.SemaphoreType.DMA((NBUF,)),
          pltpu.SemaphoreType.DMA((NBUF,)),
      ],
  )
  def sc_scatter(idx_hbm, m0_hbm, m1_hbm, m2_hbm, z_hbm,
                 a0_hbm, a1a_hbm, a1b_hbm, a2_hbm,
                 ib, rows, acc, isem, gsem, ssem):
    c = lax.axis_index("c")
    s = lax.axis_index("s")

    def run(m_hbm, a_hbm, cb, ng):
      def istart(ci, ji):
        pltpu.async_copy(idx_hbm.at[s, cb + ci], ib.at[ji], isem.at[ji])

      def iwait(ci, ji):
        pltpu.make_async_copy(idx_hbm.at[s, cb + ci], ib.at[ji],
                              isem.at[ji]).wait()

      pltpu.sync_copy(z_hbm, acc.at[pl.ds(s * rp, rp)])
      plsc.subcore_barrier()

      def gstart(ji, j):
        pltpu.async_copy(m_hbm.at[ib.at[ji, 0]], rows[j], gsem.at[j])

      def gwait(ji, j):
        pltpu.make_async_copy(m_hbm.at[ib.at[ji, 0]], rows[j],
                              gsem.at[j]).wait()

      def sstart(ji, j):
        pltpu.async_copy(rows[j], acc.at[ib.at[ji, 1]], ssem.at[j],
                         add=True)

      def swait(ji, j):
        pltpu.make_async_copy(rows[j], acc.at[ib.at[ji, 1]],
                              ssem.at[j]).wait()

      for t in range(ILOOK):
        istart(t, t)
      iwait(0, 0)
      gstart(0, 0)

      def body(grp, carry):
        base = grp * U
        last = ng - 1
        for j in range(U):
          ci = base + j
          jr = j % NBUF
          jqi = j % IBUF
          jj = (j + 1) % IBUF
          jr1 = (j + 1) % NBUF
          jp = (j + ILOOK) % IBUF
          jm = (j - SLAG) % IBUF
          jrm = (j - SLAG) % NBUF

          def when_or(always, cond, fn):
            if always:
              fn()
            else:
              pl.when(cond)(fn)

          when_or(j < U - 1, grp < last, lambda: iwait(ci + 1, jj))
          when_or(j >= SLAG, grp > 0, lambda: swait(jm, jrm))
          when_or(j < U - 1, grp < last, lambda: gstart(jj, jr1))
          when_or(j < U - ILOOK, grp < last, lambda: istart(ci + ILOOK, jp))
          gwait(jqi, jr)
          sstart(jqi, jr)
        return carry

      chn = ng * U
      lax.fori_loop(0, ng, body, 0)
      for t in range(chn - SLAG, chn):
        swait(t % IBUF, t % NBUF)
      plsc.subcore_barrier()
      pltpu.sync_copy(acc.at[pl.ds(s * rp, rp)], a_hbm.at[pl.ds(s * rp, rp)])

    @pl.when(c == 0)
    def _():
      run(m0_hbm, a0_hbm, 0, ngrp)
      run(m1_hbm, a1a_hbm, 0, ngrp // 2)

    @pl.when(c == 1)
    def _():
      run(m2_hbm, a2_hbm, 0, ngrp)
      run(m1_hbm, a1b_hbm, (ngrp // 2) * U, ngrp - ngrp // 2)

  return sc_scatter


def kernel(x, edge_index, graph_ids, W_lift, b_lift, W_m1, b_m1, W_o1, b_o1,
           W_m2, b_m2, W_o2, b_o2, W_m3, b_m3, W_o3, b_o3, W_r, b_r):
  n, din = x.shape
  e = edge_index.shape[1]
  ncls = W_r.shape[1]
  nb = 10

  def pad_w(w, rows, cols):
    return jnp.pad(w, ((0, rows - w.shape[0]), (0, cols - w.shape[1])))

  def pad_b(b, cols):
    return jnp.pad(b, (0, cols - b.shape[0])).reshape(1, cols)

  wl = pad_w(W_lift, din, HP)
  bl = pad_b(b_lift, HP)
  wm = [pad_w(W_m1, HP, HP), pad_w(W_m2, HP, HP), pad_w(W_m3, HP, HP)]
  bm = [pad_b(b_m1, HP), pad_b(b_m2, HP), pad_b(b_m3, HP)]
  wo = [pad_w(W_o1, HP, HP), pad_w(W_o2, HP, HP), pad_w(W_o3, HP, HP)]
  bo = [pad_b(b_o1, HP), pad_b(b_o2, HP), pad_b(b_o3, HP)]
  wr = pad_w(W_r, HP, OUT_PAD)
  br = pad_b(b_r, OUT_PAD)

  sc_call = _make_sc_scatter(n, e)
  ew, ch = _sc_chunks(e)
  pad = ch * K - ew
  src_r = jnp.concatenate(
      [edge_index[0].reshape(NS, ew),
       jnp.zeros((NS, pad), jnp.int32)], axis=1).reshape(NS, ch, K)
  dst_r = jnp.concatenate(
      [edge_index[1].reshape(NS, ew),
       jnp.full((NS, pad), n, jnp.int32)], axis=1).reshape(NS, ch, K)
  idx_r = jnp.stack([src_r, dst_r], axis=2)
  z = jnp.zeros((_pad_rows(n) // NS, PH), jnp.float32)
  gid3 = graph_ids.reshape(n // BLK, 1, BLK)

  h0, m0, m1, m2 = _make_lift(n, din)(x, wl, bl, wm[0], bm[0])
  a0, a1a, a1b, a2 = sc_call(idx_r, m0, m1, m2, z)
  h1, m0, m1, m2 = _make_mp(n)(a0, a1a, a1b, a2, h0, wo[0], bo[0],
                               wm[1], bm[1])
  a0, a1a, a1b, a2 = sc_call(idx_r, m0, m1, m2, z)
  h2, m0, m1, m2 = _make_mp(n)(a0, a1a, a1b, a2, h1, wo[1], bo[1],
                               wm[2], bm[2])
  a0, a1a, a1b, a2 = sc_call(idx_r, m0, m1, m2, z)
  out = _make_final(n)(a0, a1a, a1b, a2, h2, wo[2], bo[2], wr, br, gid3)
  return out[:nb, :ncls]


# speedup vs baseline: 2.7247x; 2.7247x over previous
import functools

import jax
import jax.numpy as jnp
from jax import lax
from jax.experimental import pallas as pl
from jax.experimental.pallas import tpu as pltpu
from jax.experimental.pallas import tpu_sc as plsc

HP = 384
PH = 128
ONES_COL = 310
ONES_LOCAL = ONES_COL - 2 * PH
NS = 16
BLK = 2000
OUT_PAD = 128


def _mm(a, b):
  return lax.dot_general(a, b, (((1,), (0,)), ((), ())),
                         preferred_element_type=jnp.float32,
                         precision=lax.Precision.HIGHEST)


def _set_ones_col(m):
  col = lax.broadcasted_iota(jnp.int32, m.shape, 1)
  return jnp.where(col == ONES_COL, 1.0, m)


def _lift_body(x_ref, wl_ref, bl_ref, wm_ref, bm_ref,
               h_ref, m0_ref, m1_ref, m2_ref):
  h = _mm(x_ref[...], wl_ref[...]) + bl_ref[...]
  h_ref[...] = h
  m = jnp.maximum(_mm(h, wm_ref[...]) + bm_ref[...], 0.0)
  m = _set_ones_col(m)
  m0_ref[...] = m[:, :PH]
  m1_ref[...] = m[:, PH:2 * PH]
  m2_ref[...] = m[:, 2 * PH:]


def _mp_body(a0_ref, a1a_ref, a1b_ref, a2_ref, hp_ref, wo_ref, bo_ref,
             wm_ref, bm_ref, h_ref, m0_ref, m1_ref, m2_ref):
  deg = a2_ref[:, ONES_LOCAL:ONES_LOCAL + 1]
  a1 = a1a_ref[...] + a1b_ref[...]
  feats = jnp.concatenate([a0_ref[...], a1, a2_ref[...]], axis=1)
  feats = jnp.where(deg > 0.0, feats, hp_ref[...])
  h = jnp.maximum(_mm(feats, wo_ref[...]) + bo_ref[...], 0.0)
  h_ref[...] = h
  m = jnp.maximum(_mm(h, wm_ref[...]) + bm_ref[...], 0.0)
  m = _set_ones_col(m)
  m0_ref[...] = m[:, :PH]
  m1_ref[...] = m[:, PH:2 * PH]
  m2_ref[...] = m[:, 2 * PH:]


def _final_body(a0_ref, a1a_ref, a1b_ref, a2_ref, hp_ref, wo_ref, bo_ref,
                wr_ref, br_ref, gid_ref, out_ref):
  i = pl.program_id(0)
  deg = a2_ref[:, ONES_LOCAL:ONES_LOCAL + 1]
  a1 = a1a_ref[...] + a1b_ref[...]
  feats = jnp.concatenate([a0_ref[...], a1, a2_ref[...]], axis=1)
  feats = jnp.where(deg > 0.0, feats, hp_ref[...])
  h = jnp.maximum(_mm(feats, wo_ref[...]) + bo_ref[...], 0.0)
  lg = _mm(h, wr_ref[...]) + br_ref[...]
  col = lax.broadcasted_iota(jnp.int32, lg.shape, 1)
  is_cls = col < 2
  mx = jnp.max(jnp.where(is_cls, lg, -1e30), axis=1, keepdims=True)
  ex = jnp.where(is_cls, jnp.exp(lg - mx), 0.0)
  p = ex / jnp.sum(ex, axis=1, keepdims=True)
  gid = gid_ref[0, 0, :]
  onehot = (gid[:, None] == col).astype(jnp.float32)
  contrib = lax.dot_general(onehot, p, (((0,), (0,)), ((), ())),
                            preferred_element_type=jnp.float32,
                            precision=lax.Precision.HIGHEST)

  @pl.when(i == 0)
  def _():
    out_ref[...] = jnp.zeros_like(out_ref)

  out_ref[...] += contrib


def _row_spec(blk, w):
  return pl.BlockSpec((blk, w), lambda i: (i, 0))


def _full_spec(r, c):
  return pl.BlockSpec((r, c), lambda i: (0, 0))


def _node_outs(n):
  return (
      [_row_spec(BLK, HP)] + [_row_spec(BLK, PH)] * 3,
      [jax.ShapeDtypeStruct((n, HP), jnp.float32)] +
      [jax.ShapeDtypeStruct((n, PH), jnp.float32)] * 3,
  )


def _make_lift(n, din):
  out_specs, out_shape = _node_outs(n)
  return pl.pallas_call(
      _lift_body,
      grid=(n // BLK,),
      in_specs=[_row_spec(BLK, din), _full_spec(din, HP), _full_spec(1, HP),
                _full_spec(HP, HP), _full_spec(1, HP)],
      out_specs=out_specs,
      out_shape=out_shape,
  )


def _make_mp(n):
  out_specs, out_shape = _node_outs(n)
  return pl.pallas_call(
      _mp_body,
      grid=(n // BLK,),
      in_specs=[_row_spec(BLK, PH)] * 4 + [_row_spec(BLK, HP),
                _full_spec(HP, HP), _full_spec(1, HP),
                _full_spec(HP, HP), _full_spec(1, HP)],
      out_specs=out_specs,
      out_shape=out_shape,
  )


def _make_final(n):
  return pl.pallas_call(
      _final_body,
      grid=(n // BLK,),
      in_specs=[_row_spec(BLK, PH)] * 4 + [_row_spec(BLK, HP),
                _full_spec(HP, HP), _full_spec(1, HP),
                _full_spec(HP, OUT_PAD), _full_spec(1, OUT_PAD),
                pl.BlockSpec((1, 1, BLK), lambda i: (i, 0, 0))],
      out_specs=pl.BlockSpec((OUT_PAD, OUT_PAD), lambda i: (0, 0)),
      out_shape=jax.ShapeDtypeStruct((OUT_PAD, OUT_PAD), jnp.float32),
  )


def _pad_rows(n):
  return ((n + 8 * NS - 1) // (8 * NS)) * (8 * NS)


K = 120
NBUF = 3
IBUF = 6
U = 6
SLAG = NBUF - 1
ILOOK = IBUF - SLAG


def _sc_chunks(e):
  ew = e // NS
  ch = -(-ew // K)
  ch = -(-ch // (2 * U)) * (2 * U)
  return ew, ch


def _make_sc_scatter(n, e):
  ew, ch = _sc_chunks(e)
  ngrp = ch // U
  np_ = _pad_rows(n)
  rp = np_ // NS

  mesh = plsc.VectorSubcoreMesh(core_axis_name="c", subcore_axis_name="s")

  @functools.partial(
      pl.kernel,
      out_type=(jax.ShapeDtypeStruct((np_, PH), jnp.float32),) * 4,
      mesh=mesh,
      scratch_types=[
          pltpu.VMEM((IBUF, 2, K), jnp.int32),
          [pltpu.VMEM((K, PH), jnp.float32) for _ in range(NBUF)],
          pltpu.VMEM_SHARED((np_, PH), jnp.float32),
          pltpu.SemaphoreType.DMA((IBUF,)),
          pltpu.SemaphoreType.DMA((NBUF,)),
          pltpu.SemaphoreType.DMA((NBUF,)),
      ],
  )
  def sc_scatter(idx_hbm, m0_hbm, m1_hbm, m2_hbm, z_hbm,
                 a0_hbm, a1a_hbm, a1b_hbm, a2_hbm,
                 ib, rows, acc, isem, gsem, ssem):
    c = lax.axis_index("c")
    s = lax.axis_index("s")

    def run(m_hbm, a_hbm, cb, ng):
      def istart(ci, ji):
        pltpu.async_copy(idx_hbm.at[s, cb + ci], ib.at[ji], isem.at[ji])

      def iwait(ci, ji):
        pltpu.make_async_copy(idx_hbm.at[s, cb + ci], ib.at[ji],
                              isem.at[ji]).wait()

      pltpu.sync_copy(z_hbm, acc.at[pl.ds(s * rp, rp)])
      plsc.subcore_barrier()

      def gstart(ji, j):
        pltpu.async_copy(m_hbm.at[ib.at[ji, 0]], rows[j], gsem.at[j])

      def gwait(ji, j):
        pltpu.make_async_copy(m_hbm.at[ib.at[ji, 0]], rows[j],
                              gsem.at[j]).wait()

      def sstart(ji, j):
        pltpu.async_copy(rows[j], acc.at[ib.at[ji, 1]], ssem.at[j],
                         add=True)

      def swait(ji, j):
        pltpu.make_async_copy(rows[j], acc.at[ib.at[ji, 1]],
                              ssem.at[j]).wait()

      for t in range(ILOOK):
        istart(t, t)
      iwait(0, 0)
      gstart(0, 0)

      def body(grp, carry):
        base = grp * U
        last = ng - 1
        for j in range(U):
          ci = base + j
          jr = j % NBUF
          jqi = j % IBUF
          jj = (j + 1) % IBUF
          jr1 = (j + 1) % NBUF
          jp = (j + ILOOK) % IBUF
          jm = (j - SLAG) % IBUF
          jrm = (j - SLAG) % NBUF

          def when_or(always, cond, fn):
            if always:
              fn()
            else:
              pl.when(cond)(fn)

          when_or(j < U - 1, grp < last, lambda: iwait(ci + 1, jj))
          when_or(j >= SLAG, grp > 0, lambda: swait(jm, jrm))
          when_or(j < U - 1, grp < last, lambda: gstart(jj, jr1))
          when_or(j < U - ILOOK, grp < last, lambda: istart(ci + ILOOK, jp))
          gwait(jqi, jr)
          sstart(jqi, jr)
        return carry

      chn = ng * U
      lax.fori_loop(0, ng, body, 0)
      for t in range(chn - SLAG, chn):
        swait(t % IBUF, t % NBUF)
      plsc.subcore_barrier()
      pltpu.sync_copy(acc.at[pl.ds(s * rp, rp)], a_hbm.at[pl.ds(s * rp, rp)])

    @pl.when(c == 0)
    def _():
      run(m0_hbm, a0_hbm, 0, ngrp)
      run(m1_hbm, a1a_hbm, 0, ngrp // 2)

    @pl.when(c == 1)
    def _():
      run(m2_hbm, a2_hbm, 0, ngrp)
      run(m1_hbm, a1b_hbm, (ngrp // 2) * U, ngrp - ngrp // 2)

  return sc_scatter


def kernel(x, edge_index, graph_ids, W_lift, b_lift, W_m1, b_m1, W_o1, b_o1,
           W_m2, b_m2, W_o2, b_o2, W_m3, b_m3, W_o3, b_o3, W_r, b_r):
  n, din = x.shape
  e = edge_index.shape[1]
  ncls = W_r.shape[1]
  nb = 10

  def pad_w(w, rows, cols):
    return jnp.pad(w, ((0, rows - w.shape[0]), (0, cols - w.shape[1])))

  def pad_b(b, cols):
    return jnp.pad(b, (0, cols - b.shape[0])).reshape(1, cols)

  wl = pad_w(W_lift, din, HP)
  bl = pad_b(b_lift, HP)
  wm = [pad_w(W_m1, HP, HP), pad_w(W_m2, HP, HP), pad_w(W_m3, HP, HP)]
  bm = [pad_b(b_m1, HP), pad_b(b_m2, HP), pad_b(b_m3, HP)]
  wo = [pad_w(W_o1, HP, HP), pad_w(W_o2, HP, HP), pad_w(W_o3, HP, HP)]
  bo = [pad_b(b_o1, HP), pad_b(b_o2, HP), pad_b(b_o3, HP)]
  wr = pad_w(W_r, HP, OUT_PAD)
  br = pad_b(b_r, OUT_PAD)

  sc_call = _make_sc_scatter(n, e)
  ew, ch = _sc_chunks(e)
  pad = ch * K - ew
  src_r = jnp.concatenate(
      [edge_index[0].reshape(NS, ew),
       jnp.zeros((NS, pad), jnp.int32)], axis=1).reshape(NS, ch, K)
  dst_r = jnp.concatenate(
      [edge_index[1].reshape(NS, ew),
       jnp.full((NS, pad), n, jnp.int32)], axis=1).reshape(NS, ch, K)
  idx_r = jnp.stack([src_r, dst_r], axis=2)
  z = jnp.zeros((_pad_rows(n) // NS, PH), jnp.float32)
  gid3 = graph_ids.reshape(n // BLK, 1, BLK)

  h0, m0, m1, m2 = _make_lift(n, din)(x, wl, bl, wm[0], bm[0])
  a0, a1a, a1b, a2 = sc_call(idx_r, m0, m1, m2, z)
  h1, m0, m1, m2 = _make_mp(n)(a0, a1a, a1b, a2, h0, wo[0], bo[0],
                               wm[1], bm[1])
  a0, a1a, a1b, a2 = sc_call(idx_r, m0, m1, m2, z)
  h2, m0, m1, m2 = _make_mp(n)(a0, a1a, a1b, a2, h1, wo[1], bo[1],
                               wm[2], bm[2])
  a0, a1a, a1b, a2 = sc_call(idx_r, m0, m1, m2, z)
  out = _make_final(n)(a0, a1a, a1b, a2, h2, wo[2], bo[2], wr, br, gid3)
  return out[:nb, :ncls]
